# baseline (speedup 1.0000x reference)
import functools

import jax
import jax.numpy as jnp
from jax import lax
from jax.experimental import pallas as pl
from jax.experimental.pallas import tpu as pltpu
from jax.experimental.pallas import tpu_sc as plsc

_NUM_CORES = 2
_NUM_SUBCORES = 16
_NUM_WORKERS = _NUM_CORES * _NUM_SUBCORES
_LANES = 16

_NB = 2
_NBUF = 4
_D = 2

_TC_NBUF = 4
_TC_D = 2


@functools.lru_cache(maxsize=None)
def _build_sc_copy(B, C, H, W):
    ch_per_w = C // _NUM_WORKERS
    n_chunks = B // _NB
    n_steps = n_chunks * ch_per_w
    assert n_steps % _NBUF == 0 and _D < _NBUF
    mesh = plsc.VectorSubcoreMesh(core_axis_name="c", subcore_axis_name="s")
    out_sds = jax.ShapeDtypeStruct((B, C, H, W), jnp.float32)

    @functools.partial(
        pl.kernel,
        mesh=mesh,
        out_type=out_sds,
        scratch_types=[
            pltpu.VMEM((C + _LANES,), jnp.float32),
            pltpu.VMEM((_LANES,), jnp.float32),
            pltpu.VMEM((_NBUF, _NB, 1, H, W), jnp.float32),
        ] + [pltpu.SemaphoreType.DMA] * (2 * _NBUF),
    )
    def sc_copy(keep_hbm, swap_hbm, bn_hbm, thr_hbm, out_hbm,
                wv, thrv, ring, *sems):
        gsems, ssems = sems[:_NBUF], sems[_NBUF:]
        wid = lax.axis_index("s") * _NUM_CORES + lax.axis_index("c")
        base = wid * ch_per_w

        pltpu.sync_copy(bn_hbm, wv.at[pl.ds(0, C)])
        pltpu.sync_copy(thr_hbm, thrv)
        thr0 = thrv[...][0]
        bufs = [ring.at[j] for j in range(_NBUF)]
        dummy_src = keep_hbm.at[pl.ds(0, _NB), pl.ds(0, 1)]
        dummy_dst = out_hbm.at[pl.ds(0, _NB), pl.ds(0, 1)]

        def step_coords(s):
            job = s // n_chunks
            c = base + job
            b0 = (s - job * n_chunks) * _NB
            return c, b0

        def start_scatter(s2, slot2):
            c2, b02 = step_coords(s2)
            pltpu.make_async_copy(dummy_src, bufs[slot2], gsems[slot2]).wait()
            pltpu.async_copy(bufs[slot2],
                             out_hbm.at[pl.ds(b02, _NB), pl.ds(c2, 1)],
                             ssems[slot2])

        def body(i, carry):
            for slot in range(_NBUF):
                s = i * _NBUF + slot
                c, b0 = step_coords(s)

                @pl.when(i >= 1)
                def _(slot=slot):
                    pltpu.make_async_copy(bufs[slot], dummy_dst,
                                          ssems[slot]).wait()

                wvec = wv[pl.ds(c, _LANES)]
                swap = jnp.abs(wvec[0]) < thr0

                @pl.when(swap)
                def _(slot=slot, c=c, b0=b0):
                    pltpu.async_copy(swap_hbm.at[pl.ds(b0, _NB), pl.ds(c, 1)],
                                     bufs[slot], gsems[slot])

                @pl.when(jnp.logical_not(swap))
                def _(slot=slot, c=c, b0=b0):
                    pltpu.async_copy(keep_hbm.at[pl.ds(b0, _NB), pl.ds(c, 1)],
                                     bufs[slot], gsems[slot])

                s2 = s - _D
                slot2 = (slot - _D) % _NBUF

                @pl.when(s2 >= 0)
                def _(s2=s2, slot2=slot2):
                    start_scatter(s2, slot2)
            return carry

        lax.fori_loop(0, n_steps // _NBUF, body, 0)

        for s2 in range(n_steps - _D, n_steps):
            start_scatter(s2, s2 % _NBUF)
        for slot in range(_NBUF):
            pltpu.make_async_copy(bufs[slot], dummy_dst, ssems[slot]).wait()

    return sc_copy


@functools.lru_cache(maxsize=None)
def _build_tc_copy(B, C, H, W):
    n_steps = C
    assert n_steps % _TC_NBUF == 0 and _TC_D < _TC_NBUF
    out_sds = jax.ShapeDtypeStruct((B, C, H, W), jnp.float32)

    def tc_copy(keep_hbm, swap_hbm, bn_s, thr_s, out_hbm, ring, *sems):
        gsems, ssems = sems[:_TC_NBUF], sems[_TC_NBUF:]
        bufs = [ring.at[j] for j in range(_TC_NBUF)]
        dummy_src = keep_hbm.at[pl.ds(0, B), pl.ds(0, 1)]
        dummy_dst = out_hbm.at[pl.ds(0, B), pl.ds(0, 1)]
        thr0 = thr_s[0]

        def start_scatter(c2, slot2):
            pltpu.make_async_copy(dummy_src, bufs[slot2], gsems[slot2]).wait()
            pltpu.async_copy(bufs[slot2],
                             out_hbm.at[pl.ds(0, B), pl.ds(c2, 1)],
                             ssems[slot2])

        def body(i, carry):
            for slot in range(_TC_NBUF):
                c = i * _TC_NBUF + slot

                @pl.when(i >= 1)
                def _(slot=slot):
                    pltpu.make_async_copy(bufs[slot], dummy_dst,
                                          ssems[slot]).wait()

                swap = jnp.abs(bn_s[c]) < thr0

                @pl.when(swap)
                def _(slot=slot, c=c):
                    pltpu.async_copy(swap_hbm.at[pl.ds(0, B), pl.ds(c, 1)],
                                     bufs[slot], gsems[slot])

                @pl.when(jnp.logical_not(swap))
                def _(slot=slot, c=c):
                    pltpu.async_copy(keep_hbm.at[pl.ds(0, B), pl.ds(c, 1)],
                                     bufs[slot], gsems[slot])

                c2 = c - _TC_D
                slot2 = (slot - _TC_D) % _TC_NBUF

                @pl.when(c2 >= 0)
                def _(c2=c2, slot2=slot2):
                    start_scatter(c2, slot2)
            return carry

        lax.fori_loop(0, n_steps // _TC_NBUF, body, 0)

        for c2 in range(n_steps - _TC_D, n_steps):
            start_scatter(c2, c2 % _TC_NBUF)
        for slot in range(_TC_NBUF):
            pltpu.make_async_copy(bufs[slot], dummy_dst, ssems[slot]).wait()

    return pl.pallas_call(
        tc_copy,
        out_shape=out_sds,
        in_specs=[
            pl.BlockSpec(memory_space=pl.ANY),
            pl.BlockSpec(memory_space=pl.ANY),
            pl.BlockSpec(memory_space=pltpu.SMEM),
            pl.BlockSpec(memory_space=pltpu.SMEM),
        ],
        out_specs=pl.BlockSpec(memory_space=pl.ANY),
        scratch_shapes=[pltpu.VMEM((_TC_NBUF, B, 1, H, W), jnp.float32)]
        + [pltpu.SemaphoreType.DMA] * (2 * _TC_NBUF),
    )


def kernel(x0, x1, bn0_weight, bn1_weight, bn_threshold):
    B, C, H, W = x0.shape
    thr_v = jnp.full((_LANES,), bn_threshold, dtype=jnp.float32)
    thr_s = jnp.full((1,), bn_threshold, dtype=jnp.float32)
    sc_fn = _build_sc_copy(B, C, H, W)
    tc_fn = _build_tc_copy(B, C, H, W)
    out0 = sc_fn(x0, x1, bn0_weight, thr_v)
    mask1 = (jnp.abs(bn1_weight) < bn_threshold)[None, :, None, None]
    out1 = jnp.where(mask1, x0, x1)
    return (out0, out1)


# speedup vs baseline: 1.1421x; 1.1421x over previous
import functools

import jax
import jax.numpy as jnp
from jax import lax
from jax.experimental import pallas as pl
from jax.experimental.pallas import tpu as pltpu
from jax.experimental.pallas import tpu_sc as plsc

_NUM_CORES = 2
_NUM_SUBCORES = 16
_NUM_WORKERS = _NUM_CORES * _NUM_SUBCORES
_LANES = 16

_NB = 4
_NBUF = 2
_D = 1

_TC_NBUF = 4
_TC_D = 2


@functools.lru_cache(maxsize=None)
def _build_sc_copy(B, C, H, W):
    ch_per_w = C // _NUM_WORKERS
    n_chunks = B // _NB
    n_steps = n_chunks * ch_per_w
    assert n_steps % _NBUF == 0 and _D < _NBUF
    mesh = plsc.VectorSubcoreMesh(core_axis_name="c", subcore_axis_name="s")
    out_sds = jax.ShapeDtypeStruct((B, C, H, W), jnp.float32)

    @functools.partial(
        pl.kernel,
        mesh=mesh,
        out_type=out_sds,
        scratch_types=[
            pltpu.VMEM((C + _LANES,), jnp.float32),
            pltpu.VMEM((_LANES,), jnp.float32),
            pltpu.VMEM((_NBUF, _NB, 1, H, W), jnp.float32),
        ] + [pltpu.SemaphoreType.DMA] * (2 * _NBUF),
    )
    def sc_copy(keep_hbm, swap_hbm, bn_hbm, thr_hbm, out_hbm,
                wv, thrv, ring, *sems):
        gsems, ssems = sems[:_NBUF], sems[_NBUF:]
        wid = lax.axis_index("s") * _NUM_CORES + lax.axis_index("c")
        base = wid * ch_per_w

        pltpu.sync_copy(bn_hbm, wv.at[pl.ds(0, C)])
        pltpu.sync_copy(thr_hbm, thrv)
        thr0 = thrv[...][0]
        bufs = [ring.at[j] for j in range(_NBUF)]
        dummy_src = keep_hbm.at[pl.ds(0, _NB), pl.ds(0, 1)]
        dummy_dst = out_hbm.at[pl.ds(0, _NB), pl.ds(0, 1)]

        def step_coords(s):
            job = s // n_chunks
            c = base + job
            b0 = (s - job * n_chunks) * _NB
            return c, b0

        def start_scatter(s2, slot2):
            c2, b02 = step_coords(s2)
            pltpu.make_async_copy(dummy_src, bufs[slot2], gsems[slot2]).wait()
            pltpu.async_copy(bufs[slot2],
                             out_hbm.at[pl.ds(b02, _NB), pl.ds(c2, 1)],
                             ssems[slot2])

        def body(i, carry):
            for slot in range(_NBUF):
                s = i * _NBUF + slot
                c, b0 = step_coords(s)

                @pl.when(i >= 1)
                def _(slot=slot):
                    pltpu.make_async_copy(bufs[slot], dummy_dst,
                                          ssems[slot]).wait()

                wvec = wv[pl.ds(c, _LANES)]
                swap = jnp.abs(wvec[0]) < thr0

                @pl.when(swap)
                def _(slot=slot, c=c, b0=b0):
                    pltpu.async_copy(swap_hbm.at[pl.ds(b0, _NB), pl.ds(c, 1)],
                                     bufs[slot], gsems[slot])

                @pl.when(jnp.logical_not(swap))
                def _(slot=slot, c=c, b0=b0):
                    pltpu.async_copy(keep_hbm.at[pl.ds(b0, _NB), pl.ds(c, 1)],
                                     bufs[slot], gsems[slot])

                s2 = s - _D
                slot2 = (slot - _D) % _NBUF

                @pl.when(s2 >= 0)
                def _(s2=s2, slot2=slot2):
                    start_scatter(s2, slot2)
            return carry

        lax.fori_loop(0, n_steps // _NBUF, body, 0)

        for s2 in range(n_steps - _D, n_steps):
            start_scatter(s2, s2 % _NBUF)
        for slot in range(_NBUF):
            pltpu.make_async_copy(bufs[slot], dummy_dst, ssems[slot]).wait()

    return sc_copy


@functools.lru_cache(maxsize=None)
def _build_tc_copy(B, C, H, W):
    n_steps = C
    assert n_steps % _TC_NBUF == 0 and _TC_D < _TC_NBUF
    out_sds = jax.ShapeDtypeStruct((B, C, H, W), jnp.float32)

    def tc_copy(keep_hbm, swap_hbm, bn_s, thr_s, out_hbm, ring, *sems):
        gsems, ssems = sems[:_TC_NBUF], sems[_TC_NBUF:]
        bufs = [ring.at[j] for j in range(_TC_NBUF)]
        dummy_src = keep_hbm.at[pl.ds(0, B), pl.ds(0, 1)]
        dummy_dst = out_hbm.at[pl.ds(0, B), pl.ds(0, 1)]
        thr0 = thr_s[0]

        def start_scatter(c2, slot2):
            pltpu.make_async_copy(dummy_src, bufs[slot2], gsems[slot2]).wait()
            pltpu.async_copy(bufs[slot2],
                             out_hbm.at[pl.ds(0, B), pl.ds(c2, 1)],
                             ssems[slot2])

        def body(i, carry):
            for slot in range(_TC_NBUF):
                c = i * _TC_NBUF + slot

                @pl.when(i >= 1)
                def _(slot=slot):
                    pltpu.make_async_copy(bufs[slot], dummy_dst,
                                          ssems[slot]).wait()

                swap = jnp.abs(bn_s[c]) < thr0

                @pl.when(swap)
                def _(slot=slot, c=c):
                    pltpu.async_copy(swap_hbm.at[pl.ds(0, B), pl.ds(c, 1)],
                                     bufs[slot], gsems[slot])

                @pl.when(jnp.logical_not(swap))
                def _(slot=slot, c=c):
                    pltpu.async_copy(keep_hbm.at[pl.ds(0, B), pl.ds(c, 1)],
                                     bufs[slot], gsems[slot])

                c2 = c - _TC_D
                slot2 = (slot - _TC_D) % _TC_NBUF

                @pl.when(c2 >= 0)
                def _(c2=c2, slot2=slot2):
                    start_scatter(c2, slot2)
            return carry

        lax.fori_loop(0, n_steps // _TC_NBUF, body, 0)

        for c2 in range(n_steps - _TC_D, n_steps):
            start_scatter(c2, c2 % _TC_NBUF)
        for slot in range(_TC_NBUF):
            pltpu.make_async_copy(bufs[slot], dummy_dst, ssems[slot]).wait()

    return pl.pallas_call(
        tc_copy,
        out_shape=out_sds,
        in_specs=[
            pl.BlockSpec(memory_space=pl.ANY),
            pl.BlockSpec(memory_space=pl.ANY),
            pl.BlockSpec(memory_space=pltpu.SMEM),
            pl.BlockSpec(memory_space=pltpu.SMEM),
        ],
        out_specs=pl.BlockSpec(memory_space=pl.ANY),
        scratch_shapes=[pltpu.VMEM((_TC_NBUF, B, 1, H, W), jnp.float32)]
        + [pltpu.SemaphoreType.DMA] * (2 * _TC_NBUF),
    )


def kernel(x0, x1, bn0_weight, bn1_weight, bn_threshold):
    B, C, H, W = x0.shape
    thr_v = jnp.full((_LANES,), bn_threshold, dtype=jnp.float32)
    thr_s = jnp.full((1,), bn_threshold, dtype=jnp.float32)
    sc_fn = _build_sc_copy(B, C, H, W)
    tc_fn = _build_tc_copy(B, C, H, W)
    out0 = sc_fn(x0, x1, bn0_weight, thr_v)
    out1 = tc_fn(x1, x0, bn1_weight, thr_s)
    return (out0, out1)


# speedup vs baseline: 1.1873x; 1.0395x over previous
import functools

import jax
import jax.numpy as jnp
from jax import lax
from jax.experimental import pallas as pl
from jax.experimental.pallas import tpu as pltpu
from jax.experimental.pallas import tpu_sc as plsc

_NUM_CORES = 2
_NUM_SUBCORES = 16
_NUM_WORKERS = _NUM_CORES * _NUM_SUBCORES
_LANES = 16

_NB = 4
_NBUF = 2
_D = 1

_TC_NBUF = 8
_TC_D = 4


@functools.lru_cache(maxsize=None)
def _build_sc_copy(B, C, H, W):
    ch_per_w = C // _NUM_WORKERS
    n_chunks = B // _NB
    n_steps = n_chunks * ch_per_w
    assert n_steps % _NBUF == 0 and _D < _NBUF
    mesh = plsc.VectorSubcoreMesh(core_axis_name="c", subcore_axis_name="s")
    out_sds = jax.ShapeDtypeStruct((B, C, H, W), jnp.float32)

    @functools.partial(
        pl.kernel,
        mesh=mesh,
        out_type=out_sds,
        scratch_types=[
            pltpu.VMEM((C + _LANES,), jnp.float32),
            pltpu.VMEM((_LANES,), jnp.float32),
            pltpu.VMEM((_NBUF, _NB, 1, H, W), jnp.float32),
        ] + [pltpu.SemaphoreType.DMA] * (2 * _NBUF),
    )
    def sc_copy(keep_hbm, swap_hbm, bn_hbm, thr_hbm, out_hbm,
                wv, thrv, ring, *sems):
        gsems, ssems = sems[:_NBUF], sems[_NBUF:]
        wid = lax.axis_index("s") * _NUM_CORES + lax.axis_index("c")
        base = wid * ch_per_w

        pltpu.sync_copy(bn_hbm, wv.at[pl.ds(0, C)])
        pltpu.sync_copy(thr_hbm, thrv)
        thr0 = thrv[...][0]
        bufs = [ring.at[j] for j in range(_NBUF)]
        dummy_src = keep_hbm.at[pl.ds(0, _NB), pl.ds(0, 1)]
        dummy_dst = out_hbm.at[pl.ds(0, _NB), pl.ds(0, 1)]

        def step_coords(s):
            job = s // n_chunks
            c = base + job
            b0 = (s - job * n_chunks) * _NB
            return c, b0

        def start_scatter(s2, slot2):
            c2, b02 = step_coords(s2)
            pltpu.make_async_copy(dummy_src, bufs[slot2], gsems[slot2]).wait()
            pltpu.async_copy(bufs[slot2],
                             out_hbm.at[pl.ds(b02, _NB), pl.ds(c2, 1)],
                             ssems[slot2])

        def body(i, carry):
            for slot in range(_NBUF):
                s = i * _NBUF + slot
                c, b0 = step_coords(s)

                @pl.when(i >= 1)
                def _(slot=slot):
                    pltpu.make_async_copy(bufs[slot], dummy_dst,
                                          ssems[slot]).wait()

                wvec = wv[pl.ds(c, _LANES)]
                swap = jnp.abs(wvec[0]) < thr0

                @pl.when(swap)
                def _(slot=slot, c=c, b0=b0):
                    pltpu.async_copy(swap_hbm.at[pl.ds(b0, _NB), pl.ds(c, 1)],
                                     bufs[slot], gsems[slot])

                @pl.when(jnp.logical_not(swap))
                def _(slot=slot, c=c, b0=b0):
                    pltpu.async_copy(keep_hbm.at[pl.ds(b0, _NB), pl.ds(c, 1)],
                                     bufs[slot], gsems[slot])

                s2 = s - _D
                slot2 = (slot - _D) % _NBUF

                @pl.when(s2 >= 0)
                def _(s2=s2, slot2=slot2):
                    start_scatter(s2, slot2)
            return carry

        lax.fori_loop(0, n_steps // _NBUF, body, 0)

        for s2 in range(n_steps - _D, n_steps):
            start_scatter(s2, s2 % _NBUF)
        for slot in range(_NBUF):
            pltpu.make_async_copy(bufs[slot], dummy_dst, ssems[slot]).wait()

    return sc_copy


@functools.lru_cache(maxsize=None)
def _build_tc_copy(B, C, H, W):
    n_steps = C
    assert n_steps % _TC_NBUF == 0 and _TC_D < _TC_NBUF
    out_sds = jax.ShapeDtypeStruct((B, C, H, W), jnp.float32)

    def tc_copy(keep_hbm, swap_hbm, bn_s, thr_s, out_hbm, ring, *sems):
        gsems, ssems = sems[:_TC_NBUF], sems[_TC_NBUF:]
        bufs = [ring.at[j] for j in range(_TC_NBUF)]
        dummy_src = keep_hbm.at[pl.ds(0, B), pl.ds(0, 1)]
        dummy_dst = out_hbm.at[pl.ds(0, B), pl.ds(0, 1)]
        thr0 = thr_s[0]

        def start_scatter(c2, slot2):
            pltpu.make_async_copy(dummy_src, bufs[slot2], gsems[slot2]).wait()
            pltpu.async_copy(bufs[slot2],
                             out_hbm.at[pl.ds(0, B), pl.ds(c2, 1)],
                             ssems[slot2])

        def body(i, carry):
            for slot in range(_TC_NBUF):
                c = i * _TC_NBUF + slot

                @pl.when(i >= 1)
                def _(slot=slot):
                    pltpu.make_async_copy(bufs[slot], dummy_dst,
                                          ssems[slot]).wait()

                swap = jnp.abs(bn_s[c]) < thr0

                @pl.when(swap)
                def _(slot=slot, c=c):
                    pltpu.async_copy(swap_hbm.at[pl.ds(0, B), pl.ds(c, 1)],
                                     bufs[slot], gsems[slot])

                @pl.when(jnp.logical_not(swap))
                def _(slot=slot, c=c):
                    pltpu.async_copy(keep_hbm.at[pl.ds(0, B), pl.ds(c, 1)],
                                     bufs[slot], gsems[slot])

                c2 = c - _TC_D
                slot2 = (slot - _TC_D) % _TC_NBUF

                @pl.when(c2 >= 0)
                def _(c2=c2, slot2=slot2):
                    start_scatter(c2, slot2)
            return carry

        lax.fori_loop(0, n_steps // _TC_NBUF, body, 0)

        for c2 in range(n_steps - _TC_D, n_steps):
            start_scatter(c2, c2 % _TC_NBUF)
        for slot in range(_TC_NBUF):
            pltpu.make_async_copy(bufs[slot], dummy_dst, ssems[slot]).wait()

    return pl.pallas_call(
        tc_copy,
        out_shape=out_sds,
        in_specs=[
            pl.BlockSpec(memory_space=pl.ANY),
            pl.BlockSpec(memory_space=pl.ANY),
            pl.BlockSpec(memory_space=pltpu.SMEM),
            pl.BlockSpec(memory_space=pltpu.SMEM),
        ],
        out_specs=pl.BlockSpec(memory_space=pl.ANY),
        scratch_shapes=[pltpu.VMEM((_TC_NBUF, B, 1, H, W), jnp.float32)]
        + [pltpu.SemaphoreType.DMA] * (2 * _TC_NBUF),
    )


def kernel(x0, x1, bn0_weight, bn1_weight, bn_threshold):
    B, C, H, W = x0.shape
    thr_v = jnp.full((_LANES,), bn_threshold, dtype=jnp.float32)
    thr_s = jnp.full((1,), bn_threshold, dtype=jnp.float32)
    sc_fn = _build_sc_copy(B, C, H, W)
    tc_fn = _build_tc_copy(B, C, H, W)
    out0 = sc_fn(x0, x1, bn0_weight, thr_v)
    out1 = tc_fn(x1, x0, bn1_weight, thr_s)
    return (out0, out1)


# speedup vs baseline: 1.1971x; 1.0083x over previous
import functools

import jax
import jax.numpy as jnp
from jax import lax
from jax.experimental import pallas as pl
from jax.experimental.pallas import tpu as pltpu
from jax.experimental.pallas import tpu_sc as plsc

_NUM_CORES = 2
_NUM_SUBCORES = 16
_NUM_WORKERS = _NUM_CORES * _NUM_SUBCORES
_LANES = 16

_NB = 2
_NBUF = 4
_D = 2

_TC_NBUF = 12
_TC_D = 6


@functools.lru_cache(maxsize=None)
def _build_sc_copy(B, C, H, W):
    ch_per_w = C // _NUM_WORKERS
    n_chunks = B // _NB
    n_steps = n_chunks * ch_per_w
    assert n_steps % _NBUF == 0 and _D < _NBUF
    mesh = plsc.VectorSubcoreMesh(core_axis_name="c", subcore_axis_name="s")
    out_sds = jax.ShapeDtypeStruct((B, C, H, W), jnp.float32)

    @functools.partial(
        pl.kernel,
        mesh=mesh,
        out_type=out_sds,
        scratch_types=[
            pltpu.VMEM((C + _LANES,), jnp.float32),
            pltpu.VMEM((_LANES,), jnp.float32),
            pltpu.VMEM((_NBUF, _NB, 1, H, W), jnp.float32),
        ] + [pltpu.SemaphoreType.DMA] * (2 * _NBUF),
    )
    def sc_copy(keep_hbm, swap_hbm, bn_hbm, thr_hbm, out_hbm,
                wv, thrv, ring, *sems):
        gsems, ssems = sems[:_NBUF], sems[_NBUF:]
        wid = lax.axis_index("s") * _NUM_CORES + lax.axis_index("c")
        base = wid * ch_per_w

        pltpu.sync_copy(bn_hbm, wv.at[pl.ds(0, C)])
        pltpu.sync_copy(thr_hbm, thrv)
        thr0 = thrv[...][0]
        bufs = [ring.at[j] for j in range(_NBUF)]
        dummy_src = keep_hbm.at[pl.ds(0, _NB), pl.ds(0, 1)]
        dummy_dst = out_hbm.at[pl.ds(0, _NB), pl.ds(0, 1)]

        def step_coords(s):
            job = s // n_chunks
            c = base + job
            b0 = (s - job * n_chunks) * _NB
            return c, b0

        def start_scatter(s2, slot2):
            c2, b02 = step_coords(s2)
            pltpu.make_async_copy(dummy_src, bufs[slot2], gsems[slot2]).wait()
            pltpu.async_copy(bufs[slot2],
                             out_hbm.at[pl.ds(b02, _NB), pl.ds(c2, 1)],
                             ssems[slot2])

        def body(i, carry):
            for slot in range(_NBUF):
                s = i * _NBUF + slot
                c, b0 = step_coords(s)

                @pl.when(i >= 1)
                def _(slot=slot):
                    pltpu.make_async_copy(bufs[slot], dummy_dst,
                                          ssems[slot]).wait()

                wvec = wv[pl.ds(c, _LANES)]
                swap = jnp.abs(wvec[0]) < thr0

                @pl.when(swap)
                def _(slot=slot, c=c, b0=b0):
                    pltpu.async_copy(swap_hbm.at[pl.ds(b0, _NB), pl.ds(c, 1)],
                                     bufs[slot], gsems[slot])

                @pl.when(jnp.logical_not(swap))
                def _(slot=slot, c=c, b0=b0):
                    pltpu.async_copy(keep_hbm.at[pl.ds(b0, _NB), pl.ds(c, 1)],
                                     bufs[slot], gsems[slot])

                s2 = s - _D
                slot2 = (slot - _D) % _NBUF

                @pl.when(s2 >= 0)
                def _(s2=s2, slot2=slot2):
                    start_scatter(s2, slot2)
            return carry

        lax.fori_loop(0, n_steps // _NBUF, body, 0)

        for s2 in range(n_steps - _D, n_steps):
            start_scatter(s2, s2 % _NBUF)
        for slot in range(_NBUF):
            pltpu.make_async_copy(bufs[slot], dummy_dst, ssems[slot]).wait()

    return sc_copy


@functools.lru_cache(maxsize=None)
def _build_tc_copy(B, C, H, W):
    n_steps = C
    assert n_steps % _TC_NBUF == 0 and _TC_D < _TC_NBUF
    out_sds = jax.ShapeDtypeStruct((B, C, H, W), jnp.float32)

    def tc_copy(keep_hbm, swap_hbm, bn_s, thr_s, out_hbm, ring, *sems):
        gsems, ssems = sems[:_TC_NBUF], sems[_TC_NBUF:]
        bufs = [ring.at[j] for j in range(_TC_NBUF)]
        dummy_src = keep_hbm.at[pl.ds(0, B), pl.ds(0, 1)]
        dummy_dst = out_hbm.at[pl.ds(0, B), pl.ds(0, 1)]
        thr0 = thr_s[0]

        def start_scatter(c2, slot2):
            pltpu.make_async_copy(dummy_src, bufs[slot2], gsems[slot2]).wait()
            pltpu.async_copy(bufs[slot2],
                             out_hbm.at[pl.ds(0, B), pl.ds(c2, 1)],
                             ssems[slot2])

        def body(i, carry):
            for slot in range(_TC_NBUF):
                c = i * _TC_NBUF + slot

                @pl.when(i >= 1)
                def _(slot=slot):
                    pltpu.make_async_copy(bufs[slot], dummy_dst,
                                          ssems[slot]).wait()

                swap = jnp.abs(bn_s[c]) < thr0

                @pl.when(swap)
                def _(slot=slot, c=c):
                    pltpu.async_copy(swap_hbm.at[pl.ds(0, B), pl.ds(c, 1)],
                                     bufs[slot], gsems[slot])

                @pl.when(jnp.logical_not(swap))
                def _(slot=slot, c=c):
                    pltpu.async_copy(keep_hbm.at[pl.ds(0, B), pl.ds(c, 1)],
                                     bufs[slot], gsems[slot])

                c2 = c - _TC_D
                slot2 = (slot - _TC_D) % _TC_NBUF

                @pl.when(c2 >= 0)
                def _(c2=c2, slot2=slot2):
                    start_scatter(c2, slot2)
            return carry

        lax.fori_loop(0, n_steps // _TC_NBUF, body, 0)

        for c2 in range(n_steps - _TC_D, n_steps):
            start_scatter(c2, c2 % _TC_NBUF)
        for slot in range(_TC_NBUF):
            pltpu.make_async_copy(bufs[slot], dummy_dst, ssems[slot]).wait()

    return pl.pallas_call(
        tc_copy,
        out_shape=out_sds,
        in_specs=[
            pl.BlockSpec(memory_space=pl.ANY),
            pl.BlockSpec(memory_space=pl.ANY),
            pl.BlockSpec(memory_space=pltpu.SMEM),
            pl.BlockSpec(memory_space=pltpu.SMEM),
        ],
        out_specs=pl.BlockSpec(memory_space=pl.ANY),
        scratch_shapes=[pltpu.VMEM((_TC_NBUF, B, 1, H, W), jnp.float32)]
        + [pltpu.SemaphoreType.DMA] * (2 * _TC_NBUF),
    )


_CB = 8


@functools.lru_cache(maxsize=None)
def _build_tc_dense(B, C, H, W):
    out_sds = jax.ShapeDtypeStruct((B, C, H, W), jnp.float32)

    def tc_dense(bn_s, thr_s, keep_ref, swap_ref, out_ref):
        cblk = pl.program_id(1)
        thr0 = thr_s[0]
        for k in range(_CB):
            flag = jnp.abs(bn_s[cblk * _CB + k]) < thr0

            @pl.when(flag)
            def _(k=k):
                out_ref[0, k] = swap_ref[0, k]

            @pl.when(jnp.logical_not(flag))
            def _(k=k):
                out_ref[0, k] = keep_ref[0, k]

    blk = pl.BlockSpec((1, _CB, H, W), lambda b, c: (b, c, 0, 0))
    return pl.pallas_call(
        tc_dense,
        grid=(B, C // _CB),
        out_shape=out_sds,
        in_specs=[
            pl.BlockSpec(memory_space=pltpu.SMEM),
            pl.BlockSpec(memory_space=pltpu.SMEM),
            blk,
            blk,
        ],
        out_specs=blk,
    )


def kernel(x0, x1, bn0_weight, bn1_weight, bn_threshold):
    B, C, H, W = x0.shape
    thr_v = jnp.full((_LANES,), bn_threshold, dtype=jnp.float32)
    thr_s = jnp.full((1,), bn_threshold, dtype=jnp.float32)
    sc_fn = _build_sc_copy(B, C, H, W)
    tc_fn = _build_tc_copy(B, C, H, W)
    out0 = sc_fn(x0, x1, bn0_weight, thr_v)
    out1 = tc_fn(x1, x0, bn1_weight, thr_s)
    return (out0, out1)


# speedup vs baseline: 1.1977x; 1.0005x over previous
import functools

import jax
import jax.numpy as jnp
from jax import lax
from jax.experimental import pallas as pl
from jax.experimental.pallas import tpu as pltpu
from jax.experimental.pallas import tpu_sc as plsc

_NUM_CORES = 2
_NUM_SUBCORES = 16
_NUM_WORKERS = _NUM_CORES * _NUM_SUBCORES
_LANES = 16

_NB = 2
_NBUF = 4
_D = 2

_TC_NBUF = 16
_TC_D = 8


@functools.lru_cache(maxsize=None)
def _build_sc_copy(B, C, H, W):
    ch_per_w = C // _NUM_WORKERS
    n_chunks = B // _NB
    n_steps = n_chunks * ch_per_w
    assert n_steps % _NBUF == 0 and _D < _NBUF
    mesh = plsc.VectorSubcoreMesh(core_axis_name="c", subcore_axis_name="s")
    out_sds = jax.ShapeDtypeStruct((B, C, H, W), jnp.float32)

    @functools.partial(
        pl.kernel,
        mesh=mesh,
        out_type=out_sds,
        scratch_types=[
            pltpu.VMEM((C + _LANES,), jnp.float32),
            pltpu.VMEM((_LANES,), jnp.float32),
            pltpu.VMEM((_NBUF, _NB, 1, H, W), jnp.float32),
        ] + [pltpu.SemaphoreType.DMA] * (2 * _NBUF),
    )
    def sc_copy(keep_hbm, swap_hbm, bn_hbm, thr_hbm, out_hbm,
                wv, thrv, ring, *sems):
        gsems, ssems = sems[:_NBUF], sems[_NBUF:]
        wid = lax.axis_index("s") * _NUM_CORES + lax.axis_index("c")
        base = wid * ch_per_w

        pltpu.sync_copy(bn_hbm, wv.at[pl.ds(0, C)])
        pltpu.sync_copy(thr_hbm, thrv)
        thr0 = thrv[...][0]
        bufs = [ring.at[j] for j in range(_NBUF)]
        dummy_src = keep_hbm.at[pl.ds(0, _NB), pl.ds(0, 1)]
        dummy_dst = out_hbm.at[pl.ds(0, _NB), pl.ds(0, 1)]

        def step_coords(s):
            job = s // n_chunks
            c = base + job
            b0 = (s - job * n_chunks) * _NB
            return c, b0

        def start_scatter(s2, slot2):
            c2, b02 = step_coords(s2)
            pltpu.make_async_copy(dummy_src, bufs[slot2], gsems[slot2]).wait()
            pltpu.async_copy(bufs[slot2],
                             out_hbm.at[pl.ds(b02, _NB), pl.ds(c2, 1)],
                             ssems[slot2])

        def body(i, carry):
            for slot in range(_NBUF):
                s = i * _NBUF + slot
                c, b0 = step_coords(s)

                @pl.when(i >= 1)
                def _(slot=slot):
                    pltpu.make_async_copy(bufs[slot], dummy_dst,
                                          ssems[slot]).wait()

                wvec = wv[pl.ds(c, _LANES)]
                swap = jnp.abs(wvec[0]) < thr0

                @pl.when(swap)
                def _(slot=slot, c=c, b0=b0):
                    pltpu.async_copy(swap_hbm.at[pl.ds(b0, _NB), pl.ds(c, 1)],
                                     bufs[slot], gsems[slot])

                @pl.when(jnp.logical_not(swap))
                def _(slot=slot, c=c, b0=b0):
                    pltpu.async_copy(keep_hbm.at[pl.ds(b0, _NB), pl.ds(c, 1)],
                                     bufs[slot], gsems[slot])

                s2 = s - _D
                slot2 = (slot - _D) % _NBUF

                @pl.when(s2 >= 0)
                def _(s2=s2, slot2=slot2):
                    start_scatter(s2, slot2)
            return carry

        lax.fori_loop(0, n_steps // _NBUF, body, 0)

        for s2 in range(n_steps - _D, n_steps):
            start_scatter(s2, s2 % _NBUF)
        for slot in range(_NBUF):
            pltpu.make_async_copy(bufs[slot], dummy_dst, ssems[slot]).wait()

    return sc_copy


@functools.lru_cache(maxsize=None)
def _build_tc_copy(B, C, H, W):
    n_steps = C
    assert n_steps % _TC_NBUF == 0 and _TC_D < _TC_NBUF
    out_sds = jax.ShapeDtypeStruct((B, C, H, W), jnp.float32)

    def tc_copy(keep_hbm, swap_hbm, bn_s, thr_s, out_hbm, ring, *sems):
        gsems, ssems = sems[:_TC_NBUF], sems[_TC_NBUF:]
        bufs = [ring.at[j] for j in range(_TC_NBUF)]
        dummy_src = keep_hbm.at[pl.ds(0, B), pl.ds(0, 1)]
        dummy_dst = out_hbm.at[pl.ds(0, B), pl.ds(0, 1)]
        thr0 = thr_s[0]

        def start_scatter(c2, slot2):
            pltpu.make_async_copy(dummy_src, bufs[slot2], gsems[slot2]).wait()
            pltpu.async_copy(bufs[slot2],
                             out_hbm.at[pl.ds(0, B), pl.ds(c2, 1)],
                             ssems[slot2])

        def body(i, carry):
            for slot in range(_TC_NBUF):
                c = i * _TC_NBUF + slot

                @pl.when(i >= 1)
                def _(slot=slot):
                    pltpu.make_async_copy(bufs[slot], dummy_dst,
                                          ssems[slot]).wait()

                swap = jnp.abs(bn_s[c]) < thr0

                @pl.when(swap)
                def _(slot=slot, c=c):
                    pltpu.async_copy(swap_hbm.at[pl.ds(0, B), pl.ds(c, 1)],
                                     bufs[slot], gsems[slot])

                @pl.when(jnp.logical_not(swap))
                def _(slot=slot, c=c):
                    pltpu.async_copy(keep_hbm.at[pl.ds(0, B), pl.ds(c, 1)],
                                     bufs[slot], gsems[slot])

                c2 = c - _TC_D
                slot2 = (slot - _TC_D) % _TC_NBUF

                @pl.when(c2 >= 0)
                def _(c2=c2, slot2=slot2):
                    start_scatter(c2, slot2)
            return carry

        lax.fori_loop(0, n_steps // _TC_NBUF, body, 0)

        for c2 in range(n_steps - _TC_D, n_steps):
            start_scatter(c2, c2 % _TC_NBUF)
        for slot in range(_TC_NBUF):
            pltpu.make_async_copy(bufs[slot], dummy_dst, ssems[slot]).wait()

    return pl.pallas_call(
        tc_copy,
        out_shape=out_sds,
        in_specs=[
            pl.BlockSpec(memory_space=pl.ANY),
            pl.BlockSpec(memory_space=pl.ANY),
            pl.BlockSpec(memory_space=pltpu.SMEM),
            pl.BlockSpec(memory_space=pltpu.SMEM),
        ],
        out_specs=pl.BlockSpec(memory_space=pl.ANY),
        scratch_shapes=[pltpu.VMEM((_TC_NBUF, B, 1, H, W), jnp.float32)]
        + [pltpu.SemaphoreType.DMA] * (2 * _TC_NBUF),
    )


_CB = 8


@functools.lru_cache(maxsize=None)
def _build_tc_dense(B, C, H, W):
    out_sds = jax.ShapeDtypeStruct((B, C, H, W), jnp.float32)

    def tc_dense(bn_s, thr_s, keep_ref, swap_ref, out_ref):
        cblk = pl.program_id(1)
        thr0 = thr_s[0]
        for k in range(_CB):
            flag = jnp.abs(bn_s[cblk * _CB + k]) < thr0

            @pl.when(flag)
            def _(k=k):
                out_ref[0, k] = swap_ref[0, k]

            @pl.when(jnp.logical_not(flag))
            def _(k=k):
                out_ref[0, k] = keep_ref[0, k]

    blk = pl.BlockSpec((1, _CB, H, W), lambda b, c: (b, c, 0, 0))
    return pl.pallas_call(
        tc_dense,
        grid=(B, C // _CB),
        out_shape=out_sds,
        in_specs=[
            pl.BlockSpec(memory_space=pltpu.SMEM),
            pl.BlockSpec(memory_space=pltpu.SMEM),
            blk,
            blk,
        ],
        out_specs=blk,
    )


def kernel(x0, x1, bn0_weight, bn1_weight, bn_threshold):
    B, C, H, W = x0.shape
    thr_v = jnp.full((_LANES,), bn_threshold, dtype=jnp.float32)
    thr_s = jnp.full((1,), bn_threshold, dtype=jnp.float32)
    sc_fn = _build_sc_copy(B, C, H, W)
    tc_fn = _build_tc_copy(B, C, H, W)
    out0 = sc_fn(x0, x1, bn0_weight, thr_v)
    out1 = tc_fn(x1, x0, bn1_weight, thr_s)
    return (out0, out1)


# speedup vs baseline: 1.2010x; 1.0027x over previous
import functools

import jax
import jax.numpy as jnp
from jax import lax
from jax.experimental import pallas as pl
from jax.experimental.pallas import tpu as pltpu
from jax.experimental.pallas import tpu_sc as plsc

_NUM_CORES = 2
_NUM_SUBCORES = 16
_NUM_WORKERS = _NUM_CORES * _NUM_SUBCORES
_LANES = 16

_NB = 1
_NBUF = 8
_D = 4

_TC_NBUF = 16
_TC_D = 8


@functools.lru_cache(maxsize=None)
def _build_sc_copy(B, C, H, W):
    ch_per_w = C // _NUM_WORKERS
    n_chunks = B // _NB
    n_steps = n_chunks * ch_per_w
    assert n_steps % _NBUF == 0 and _D < _NBUF
    mesh = plsc.VectorSubcoreMesh(core_axis_name="c", subcore_axis_name="s")
    out_sds = jax.ShapeDtypeStruct((B, C, H, W), jnp.float32)

    @functools.partial(
        pl.kernel,
        mesh=mesh,
        out_type=out_sds,
        scratch_types=[
            pltpu.VMEM((C + _LANES,), jnp.float32),
            pltpu.VMEM((_LANES,), jnp.float32),
            pltpu.VMEM((_NBUF, _NB, 1, H, W), jnp.float32),
        ] + [pltpu.SemaphoreType.DMA] * (2 * _NBUF),
    )
    def sc_copy(keep_hbm, swap_hbm, bn_hbm, thr_hbm, out_hbm,
                wv, thrv, ring, *sems):
        gsems, ssems = sems[:_NBUF], sems[_NBUF:]
        wid = lax.axis_index("s") * _NUM_CORES + lax.axis_index("c")
        base = wid * ch_per_w

        pltpu.sync_copy(bn_hbm, wv.at[pl.ds(0, C)])
        pltpu.sync_copy(thr_hbm, thrv)
        thr0 = thrv[...][0]
        bufs = [ring.at[j] for j in range(_NBUF)]
        dummy_src = keep_hbm.at[pl.ds(0, _NB), pl.ds(0, 1)]
        dummy_dst = out_hbm.at[pl.ds(0, _NB), pl.ds(0, 1)]

        def step_coords(s):
            job = s // n_chunks
            c = base + job
            b0 = (s - job * n_chunks) * _NB
            return c, b0

        def start_scatter(s2, slot2):
            c2, b02 = step_coords(s2)
            pltpu.make_async_copy(dummy_src, bufs[slot2], gsems[slot2]).wait()
            pltpu.async_copy(bufs[slot2],
                             out_hbm.at[pl.ds(b02, _NB), pl.ds(c2, 1)],
                             ssems[slot2])

        def body(i, carry):
            for slot in range(_NBUF):
                s = i * _NBUF + slot
                c, b0 = step_coords(s)

                @pl.when(i >= 1)
                def _(slot=slot):
                    pltpu.make_async_copy(bufs[slot], dummy_dst,
                                          ssems[slot]).wait()

                wvec = wv[pl.ds(c, _LANES)]
                swap = jnp.abs(wvec[0]) < thr0

                @pl.when(swap)
                def _(slot=slot, c=c, b0=b0):
                    pltpu.async_copy(swap_hbm.at[pl.ds(b0, _NB), pl.ds(c, 1)],
                                     bufs[slot], gsems[slot])

                @pl.when(jnp.logical_not(swap))
                def _(slot=slot, c=c, b0=b0):
                    pltpu.async_copy(keep_hbm.at[pl.ds(b0, _NB), pl.ds(c, 1)],
                                     bufs[slot], gsems[slot])

                s2 = s - _D
                slot2 = (slot - _D) % _NBUF

                @pl.when(s2 >= 0)
                def _(s2=s2, slot2=slot2):
                    start_scatter(s2, slot2)
            return carry

        lax.fori_loop(0, n_steps // _NBUF, body, 0)

        for s2 in range(n_steps - _D, n_steps):
            start_scatter(s2, s2 % _NBUF)
        for slot in range(_NBUF):
            pltpu.make_async_copy(bufs[slot], dummy_dst, ssems[slot]).wait()

    return sc_copy


@functools.lru_cache(maxsize=None)
def _build_tc_copy(B, C, H, W):
    n_steps = C
    assert n_steps % _TC_NBUF == 0 and _TC_D < _TC_NBUF
    out_sds = jax.ShapeDtypeStruct((B, C, H, W), jnp.float32)

    def tc_copy(keep_hbm, swap_hbm, bn_s, thr_s, out_hbm, ring, *sems):
        gsems, ssems = sems[:_TC_NBUF], sems[_TC_NBUF:]
        bufs = [ring.at[j] for j in range(_TC_NBUF)]
        dummy_src = keep_hbm.at[pl.ds(0, B), pl.ds(0, 1)]
        dummy_dst = out_hbm.at[pl.ds(0, B), pl.ds(0, 1)]
        thr0 = thr_s[0]

        def start_scatter(c2, slot2):
            pltpu.make_async_copy(dummy_src, bufs[slot2], gsems[slot2]).wait()
            pltpu.async_copy(bufs[slot2],
                             out_hbm.at[pl.ds(0, B), pl.ds(c2, 1)],
                             ssems[slot2])

        def body(i, carry):
            for slot in range(_TC_NBUF):
                c = i * _TC_NBUF + slot

                @pl.when(i >= 1)
                def _(slot=slot):
                    pltpu.make_async_copy(bufs[slot], dummy_dst,
                                          ssems[slot]).wait()

                swap = jnp.abs(bn_s[c]) < thr0

                @pl.when(swap)
                def _(slot=slot, c=c):
                    pltpu.async_copy(swap_hbm.at[pl.ds(0, B), pl.ds(c, 1)],
                                     bufs[slot], gsems[slot])

                @pl.when(jnp.logical_not(swap))
                def _(slot=slot, c=c):
                    pltpu.async_copy(keep_hbm.at[pl.ds(0, B), pl.ds(c, 1)],
                                     bufs[slot], gsems[slot])

                c2 = c - _TC_D
                slot2 = (slot - _TC_D) % _TC_NBUF

                @pl.when(c2 >= 0)
                def _(c2=c2, slot2=slot2):
                    start_scatter(c2, slot2)
            return carry

        lax.fori_loop(0, n_steps // _TC_NBUF, body, 0)

        for c2 in range(n_steps - _TC_D, n_steps):
            start_scatter(c2, c2 % _TC_NBUF)
        for slot in range(_TC_NBUF):
            pltpu.make_async_copy(bufs[slot], dummy_dst, ssems[slot]).wait()

    return pl.pallas_call(
        tc_copy,
        out_shape=out_sds,
        in_specs=[
            pl.BlockSpec(memory_space=pl.ANY),
            pl.BlockSpec(memory_space=pl.ANY),
            pl.BlockSpec(memory_space=pltpu.SMEM),
            pl.BlockSpec(memory_space=pltpu.SMEM),
        ],
        out_specs=pl.BlockSpec(memory_space=pl.ANY),
        scratch_shapes=[pltpu.VMEM((_TC_NBUF, B, 1, H, W), jnp.float32)]
        + [pltpu.SemaphoreType.DMA] * (2 * _TC_NBUF),
    )


_CB = 8


@functools.lru_cache(maxsize=None)
def _build_tc_dense(B, C, H, W):
    out_sds = jax.ShapeDtypeStruct((B, C, H, W), jnp.float32)

    def tc_dense(bn_s, thr_s, keep_ref, swap_ref, out_ref):
        cblk = pl.program_id(1)
        thr0 = thr_s[0]
        for k in range(_CB):
            flag = jnp.abs(bn_s[cblk * _CB + k]) < thr0

            @pl.when(flag)
            def _(k=k):
                out_ref[0, k] = swap_ref[0, k]

            @pl.when(jnp.logical_not(flag))
            def _(k=k):
                out_ref[0, k] = keep_ref[0, k]

    blk = pl.BlockSpec((1, _CB, H, W), lambda b, c: (b, c, 0, 0))
    return pl.pallas_call(
        tc_dense,
        grid=(B, C // _CB),
        out_shape=out_sds,
        in_specs=[
            pl.BlockSpec(memory_space=pltpu.SMEM),
            pl.BlockSpec(memory_space=pltpu.SMEM),
            blk,
            blk,
        ],
        out_specs=blk,
    )


def kernel(x0, x1, bn0_weight, bn1_weight, bn_threshold):
    B, C, H, W = x0.shape
    thr_v = jnp.full((_LANES,), bn_threshold, dtype=jnp.float32)
    thr_s = jnp.full((1,), bn_threshold, dtype=jnp.float32)
    sc_fn = _build_sc_copy(B, C, H, W)
    tc_fn = _build_tc_copy(B, C, H, W)
    out0 = sc_fn(x0, x1, bn0_weight, thr_v)
    out1 = tc_fn(x1, x0, bn1_weight, thr_s)
    return (out0, out1)
